# baseline (speedup 1.0000x reference)
import functools

import jax
import jax.numpy as jnp
from jax import lax
from jax.experimental import pallas as pl
from jax.experimental.pallas import tpu as pltpu
from jax.experimental.pallas import tpu_sc as plsc

CYCLE = 168
T = 336
C = 64
B = 4096

_info = plsc.get_sparse_core_info()
_NC = _info.num_cores
_NS = _info.num_subcores
_NW = _NC * _NS
_BPW = B // _NW


@functools.partial(
    pl.kernel,
    mesh=plsc.VectorSubcoreMesh(core_axis_name="c", subcore_axis_name="s"),
    out_type=jax.ShapeDtypeStruct((B, T, C), jnp.float32),
    scratch_types=[
        pltpu.VMEM((3 * CYCLE, C), jnp.float32),
        pltpu.VMEM((_BPW,), jnp.int32),
        pltpu.SemaphoreType.DMA,
    ],
)
def _cycle_gather(idx_hbm, data_hbm, out_hbm, table_v, idx_v, sem):
    wid = lax.axis_index("s") * _NC + lax.axis_index("c")
    base = wid * _BPW

    pltpu.sync_copy(data_hbm, table_v.at[pl.ds(0, CYCLE)])
    pltpu.sync_copy(data_hbm, table_v.at[pl.ds(CYCLE, CYCLE)])
    pltpu.sync_copy(data_hbm, table_v.at[pl.ds(2 * CYCLE, CYCLE)])
    pltpu.sync_copy(idx_hbm.at[pl.ds(base, _BPW)], idx_v)

    def group(g, carry):
        vec = idx_v[pl.ds(pl.multiple_of(g * _NS, _NS), _NS)]
        for j in range(_NS):
            start = vec[j]
            pltpu.make_async_copy(
                table_v.at[pl.ds(start, T)], out_hbm.at[base + g * _NS + j], sem
            ).start()

        @pl.when(g > 0)
        def _():
            for j in range(_NS):
                pltpu.make_async_copy(
                    table_v.at[pl.ds(0, T)], out_hbm.at[base + j], sem
                ).wait()

        return carry

    lax.fori_loop(0, _BPW // _NS, group, 0)
    for j in range(_NS):
        pltpu.make_async_copy(table_v.at[pl.ds(0, T)], out_hbm.at[base + j], sem).wait()


def _tc_body(idx_ref, table_ref, out_ref):
    b = pl.program_id(0)
    start = idx_ref[b]
    out_ref[0] = table_ref[pl.ds(start, T), :]


def _tc_cycle(index, table3):
    return pl.pallas_call(
        _tc_body,
        grid_spec=pltpu.PrefetchScalarGridSpec(
            num_scalar_prefetch=1,
            grid=(B,),
            in_specs=[pl.BlockSpec((3 * CYCLE, C), lambda b, idx: (0, 0))],
            out_specs=pl.BlockSpec((1, T, C), lambda b, idx: (b, 0, 0)),
        ),
        out_shape=jax.ShapeDtypeStruct((B, T, C), jnp.float32),
    )(index, table3)


def kernel(index, length, data):
    del length
    table3 = jnp.concatenate([data, data, data], axis=0)
    return _tc_cycle(index.astype(jnp.int32), table3)


# speedup vs baseline: 2.4078x; 2.4078x over previous
import functools

import jax
import jax.numpy as jnp
from jax import lax
from jax.experimental import pallas as pl
from jax.experimental.pallas import tpu as pltpu
from jax.experimental.pallas import tpu_sc as plsc

CYCLE = 168
T = 336
C = 64
B = 4096

_info = plsc.get_sparse_core_info()
_NC = _info.num_cores
_NS = _info.num_subcores
_NW = _NC * _NS
_BPW = B // _NW


@functools.partial(
    pl.kernel,
    mesh=plsc.VectorSubcoreMesh(core_axis_name="c", subcore_axis_name="s"),
    out_type=jax.ShapeDtypeStruct((B, T, C), jnp.float32),
    scratch_types=[
        pltpu.VMEM((3 * CYCLE, C), jnp.float32),
        pltpu.VMEM((_BPW,), jnp.int32),
        pltpu.SemaphoreType.DMA,
    ],
)
def _cycle_gather(idx_hbm, data_hbm, out_hbm, table_v, idx_v, sem):
    wid = lax.axis_index("s") * _NC + lax.axis_index("c")
    base = wid * _BPW

    pltpu.sync_copy(data_hbm, table_v.at[pl.ds(0, CYCLE)])
    pltpu.sync_copy(data_hbm, table_v.at[pl.ds(CYCLE, CYCLE)])
    pltpu.sync_copy(data_hbm, table_v.at[pl.ds(2 * CYCLE, CYCLE)])
    pltpu.sync_copy(idx_hbm.at[pl.ds(base, _BPW)], idx_v)

    def group(g, carry):
        vec = idx_v[pl.ds(pl.multiple_of(g * _NS, _NS), _NS)]
        for j in range(_NS):
            start = vec[j]
            pltpu.make_async_copy(
                table_v.at[pl.ds(start, T)], out_hbm.at[base + g * _NS + j], sem
            ).start()

        @pl.when(g > 0)
        def _():
            for j in range(_NS):
                pltpu.make_async_copy(
                    table_v.at[pl.ds(0, T)], out_hbm.at[base + j], sem
                ).wait()

        return carry

    lax.fori_loop(0, _BPW // _NS, group, 0)
    for j in range(_NS):
        pltpu.make_async_copy(table_v.at[pl.ds(0, T)], out_hbm.at[base + j], sem).wait()


_G = 8


def _tc_body(idx_ref, table_ref, out_ref):
    b = pl.program_id(0)
    for j in range(_G):
        start = idx_ref[b * _G + j]
        out_ref[j] = table_ref[pl.ds(start, T), :]


def _tc_cycle(index, table3):
    return pl.pallas_call(
        _tc_body,
        grid_spec=pltpu.PrefetchScalarGridSpec(
            num_scalar_prefetch=1,
            grid=(B // _G,),
            in_specs=[pl.BlockSpec((3 * CYCLE, C), lambda b, idx: (0, 0))],
            out_specs=pl.BlockSpec((_G, T, C), lambda b, idx: (b, 0, 0)),
        ),
        out_shape=jax.ShapeDtypeStruct((B, T, C), jnp.float32),
    )(index, table3)


def kernel(index, length, data):
    del length
    table3 = jnp.concatenate([data, data, data], axis=0)
    return _tc_cycle(index.astype(jnp.int32), table3)


# speedup vs baseline: 2.7282x; 1.1331x over previous
import functools

import jax
import jax.numpy as jnp
from jax import lax
from jax.experimental import pallas as pl
from jax.experimental.pallas import tpu as pltpu
from jax.experimental.pallas import tpu_sc as plsc

CYCLE = 168
T = 336
C = 64
B = 4096

_info = plsc.get_sparse_core_info()
_NC = _info.num_cores
_NS = _info.num_subcores
_NW = _NC * _NS
_BPW = B // _NW


@functools.partial(
    pl.kernel,
    mesh=plsc.VectorSubcoreMesh(core_axis_name="c", subcore_axis_name="s"),
    out_type=jax.ShapeDtypeStruct((B, T, C), jnp.float32),
    scratch_types=[
        pltpu.VMEM((3 * CYCLE, C), jnp.float32),
        pltpu.VMEM((_BPW,), jnp.int32),
        pltpu.SemaphoreType.DMA,
    ],
)
def _cycle_gather(idx_hbm, data_hbm, out_hbm, table_v, idx_v, sem):
    wid = lax.axis_index("s") * _NC + lax.axis_index("c")
    base = wid * _BPW

    pltpu.sync_copy(data_hbm, table_v.at[pl.ds(0, CYCLE)])
    pltpu.sync_copy(data_hbm, table_v.at[pl.ds(CYCLE, CYCLE)])
    pltpu.sync_copy(data_hbm, table_v.at[pl.ds(2 * CYCLE, CYCLE)])
    pltpu.sync_copy(idx_hbm.at[pl.ds(base, _BPW)], idx_v)

    def group(g, carry):
        vec = idx_v[pl.ds(pl.multiple_of(g * _NS, _NS), _NS)]
        for j in range(_NS):
            start = vec[j]
            pltpu.make_async_copy(
                table_v.at[pl.ds(start, T)], out_hbm.at[base + g * _NS + j], sem
            ).start()

        @pl.when(g > 0)
        def _():
            for j in range(_NS):
                pltpu.make_async_copy(
                    table_v.at[pl.ds(0, T)], out_hbm.at[base + j], sem
                ).wait()

        return carry

    lax.fori_loop(0, _BPW // _NS, group, 0)
    for j in range(_NS):
        pltpu.make_async_copy(table_v.at[pl.ds(0, T)], out_hbm.at[base + j], sem).wait()


_G = 32


def _tc_body(idx_ref, table_ref, out_ref):
    b = pl.program_id(0)
    for j in range(_G):
        start = idx_ref[b * _G + j]
        out_ref[j] = table_ref[pl.ds(start, T), :]


def _tc_cycle(index, table3):
    return pl.pallas_call(
        _tc_body,
        grid_spec=pltpu.PrefetchScalarGridSpec(
            num_scalar_prefetch=1,
            grid=(B // _G,),
            in_specs=[pl.BlockSpec((3 * CYCLE, C), lambda b, idx: (0, 0))],
            out_specs=pl.BlockSpec((_G, T, C), lambda b, idx: (b, 0, 0)),
        ),
        out_shape=jax.ShapeDtypeStruct((B, T, C), jnp.float32),
    )(index, table3)


def kernel(index, length, data):
    del length
    table3 = jnp.concatenate([data, data, data], axis=0)
    return _tc_cycle(index.astype(jnp.int32), table3)
